# baseline (speedup 1.0000x reference)
import functools

import jax
import jax.numpy as jnp
from jax import lax
from jax.experimental import pallas as pl
from jax.experimental.pallas import tpu as pltpu
from jax.experimental.pallas import tpu_sc as plsc

_NUM_CORES = 2
_NUM_SUBCORES = 16
_NW = _NUM_CORES * _NUM_SUBCORES
_LANES = 16


def _make_sc_bag_sum(B, Lb, D, npairs, *, cb=2, nbuf=2, interpret=False):
  assert B % _NW == 0
  bpw = B // _NW
  assert bpw % cb == 0
  nchunk = bpw // cb
  assert nchunk % nbuf == 0
  assert D % _LANES == 0
  nq = D // _LANES
  ngrp = (Lb + _LANES - 1) // _LANES
  nnz_w = bpw * Lb
  nidx = cb * Lb
  nidx8 = -(-nidx // 8) * 8
  nidx16 = -(-nidx8 // _LANES) * _LANES
  assert nidx8 <= 128

  mesh = plsc.VectorSubcoreMesh(
      core_axis_name="c", subcore_axis_name="s",
      num_cores=_NUM_CORES, num_subcores=_NUM_SUBCORES)

  def body(idx_hbm, w_hbm, table_hbm, out_hbm,
           idx_v, w_v, pairs_v, rows_v, out_v, *sems):
    cid = lax.axis_index("c")
    sid = lax.axis_index("s")
    wid = sid * _NUM_CORES + cid
    bag0 = wid * bpw

    pltpu.sync_copy(idx_hbm.at[pl.ds(bag0 * Lb, nnz_w)],
                    idx_v.at[pl.ds(0, nnz_w)])
    pltpu.sync_copy(w_hbm.at[pl.ds(bag0 * Lb, nnz_w)],
                    w_v.at[pl.ds(0, nnz_w)])

    lane = lax.iota(jnp.int32, _LANES)

    def start(chunk, b):
      for g in range(nidx16 // _LANES):
        off = chunk * nidx + g * _LANES
        pv = lax.shift_right_logical(idx_v[pl.ds(off, _LANES)], 1)
        nvalid = nidx - g * _LANES
        if nvalid < _LANES:
          pv = jnp.where(lane < nvalid, pv, 0)
        pairs_v[b, pl.ds(g * _LANES, _LANES)] = pv
      pltpu.async_copy(
          table_hbm.at[pairs_v.at[b, pl.ds(0, nidx8)]],
          rows_v.at[b],
          sems[b])

    def drain(b):
      pltpu.make_async_copy(
          table_hbm.at[pairs_v.at[b, pl.ds(0, nidx8)]],
          rows_v.at[b],
          sems[b]).wait()

    for b in range(nbuf):
      start(b, b)

    def outer(i, carry):
      for b in range(nbuf):
        chunk = i * nbuf + b
        drain(b)
        for k in range(cb):
          woff = chunk * (cb * Lb) + k * Lb
          acc = [jnp.zeros((_LANES,), jnp.float32) for _ in range(nq)]
          for g in range(ngrp):
            nrows = min(_LANES, Lb - g * _LANES)
            wvec = w_v[pl.ds(woff + g * _LANES, _LANES)]
            ivec = idx_v[pl.ds(woff + g * _LANES, _LANES)]
            for j2 in range(nrows):
              wv = jnp.full((_LANES,), wvec[j2])
              colb = pl.multiple_of((ivec[j2] & 1) * D, D)
              r = k * Lb + g * _LANES + j2
              for q in range(nq):
                acc[q] = acc[q] + wv * rows_v[b, r,
                                              pl.ds(colb + q * _LANES, _LANES)]
          for q in range(nq):
            out_v[k, pl.ds(q * _LANES, _LANES)] = acc[q]
        pltpu.sync_copy(out_v, out_hbm.at[pl.ds(bag0 + chunk * cb, cb)])
        nxt = chunk + nbuf

        @pl.when(nxt < nchunk)
        def _():
          start(nxt, b)
      return carry

    lax.fori_loop(0, nchunk // nbuf, outer, 0)

  fn = pl.kernel(
      body,
      out_type=jax.ShapeDtypeStruct((B, D), jnp.float32),
      mesh=mesh,
      scratch_types=[
          pltpu.VMEM((nnz_w + _LANES,), jnp.int32),
          pltpu.VMEM((nnz_w + _LANES,), jnp.float32),
          pltpu.VMEM((nbuf, nidx16), jnp.int32),
          pltpu.VMEM((nbuf, nidx8, 2 * D), jnp.float32),
          pltpu.VMEM((cb, D), jnp.float32),
      ] + [pltpu.SemaphoreType.DMA] * nbuf,
      compiler_params=pltpu.CompilerParams(use_tc_tiling_on_sc=False),
      interpret=interpret,
  )
  return fn


def _make_tc_mlp(B, D, E, *, blk=1024, interpret=False):
  assert B % blk == 0

  def body(x_ref, w1_ref, b1_ref, w2_ref, b2_ref, o_ref):
    x = x_ref[...]
    s = jnp.sum(x * x, axis=1, keepdims=True)
    x = x / jnp.maximum(jnp.sqrt(s), 1e-12)
    h = lax.dot_general(x, w1_ref[...], (((1,), (1,)), ((), ())),
                        preferred_element_type=jnp.float32) + b1_ref[...]
    h = jnp.where(h >= 0, h, 0.01 * h)
    o_ref[...] = lax.dot_general(h, w2_ref[...], (((1,), (1,)), ((), ())),
                                 preferred_element_type=jnp.float32) + b2_ref[...]

  grid = (B // blk,)
  return pl.pallas_call(
      body,
      grid=grid,
      in_specs=[
          pl.BlockSpec((blk, E), lambda i: (i, 0)),
          pl.BlockSpec((D, E), lambda i: (0, 0)),
          pl.BlockSpec((1, D), lambda i: (0, 0)),
          pl.BlockSpec((D, D), lambda i: (0, 0)),
          pl.BlockSpec((1, D), lambda i: (0, 0)),
      ],
      out_specs=pl.BlockSpec((blk, D), lambda i: (i, 0)),
      out_shape=jax.ShapeDtypeStruct((B, D), jnp.float32),
      interpret=interpret,
  )


@jax.jit
def kernel(indices, offsets, weights, base_emb, W1, b1, W2, b2):
  del offsets
  B = 16384
  Lb = 50
  V, E = base_emb.shape
  D = W1.shape[0]
  table_pairs = base_emb.reshape(V // 2, 2 * E)
  sc = _make_sc_bag_sum(B, Lb, E, V // 2)
  bag_sums = sc(indices, weights, table_pairs)
  mlp = _make_tc_mlp(B, D, E)
  return mlp(bag_sums, W1, b1.reshape(1, D), W2, b2.reshape(1, D))


# speedup vs baseline: 3.6840x; 3.6840x over previous
import functools

import jax
import jax.numpy as jnp
from jax import lax
from jax.experimental import pallas as pl
from jax.experimental.pallas import tpu as pltpu
from jax.experimental.pallas import tpu_sc as plsc

_NUM_CORES = 2
_NUM_SUBCORES = 16
_NW = _NUM_CORES * _NUM_SUBCORES
_LANES = 16


def _make_sc_bag_sum(B, Lb, D, npairs, *, cb=8, nbuf=2, niw=4,
                     interpret=False):
  assert B % _NW == 0
  bpw = B // _NW
  assert bpw % cb == 0
  nchunk = bpw // cb
  assert nchunk % nbuf == 0 and nbuf == 2
  assert D % _LANES == 0
  nq = D // _LANES
  ngrp = (Lb + _LANES - 1) // _LANES
  nidx = cb * Lb
  assert nidx % _LANES == 0
  slot = nidx + 2 * _LANES
  wslot = nidx + _LANES
  splits = []
  off = 0
  while off < nidx:
    n = min(128 - 128 % 8, nidx - off)
    n -= n % 8
    splits.append((off, n))
    off += n
  assert sum(n for _, n in splits) == nidx

  mesh = plsc.VectorSubcoreMesh(
      core_axis_name="c", subcore_axis_name="s",
      num_cores=_NUM_CORES, num_subcores=_NUM_SUBCORES)

  def body(idx_hbm, w_hbm, table_hbm, out_hbm,
           idxr_v, wr_v, pairs_v, wlo_v, whi_v, rows_v, out_v, *sems):
    gsems = sems[:nbuf]
    iwsems = sems[nbuf:]
    cid = lax.axis_index("c")
    sid = lax.axis_index("s")
    wid = sid * _NUM_CORES + cid
    gbase = wid * bpw * Lb
    bag0 = wid * bpw

    def iw_copies(c, s):
      return (
          pltpu.make_async_copy(
              idx_hbm.at[pl.ds(gbase + c * nidx, nidx)],
              idxr_v.at[pl.ds(s * slot, nidx)], iwsems[s]),
          pltpu.make_async_copy(
              w_hbm.at[pl.ds(gbase + c * nidx, nidx)],
              wr_v.at[pl.ds(s * slot, nidx)], iwsems[s]),
      )

    def issue_iw(c, s):
      for d in iw_copies(c, s):
        d.start()

    def wait_iw(c, s):
      for d in iw_copies(c, s):
        d.wait()

    def prep(c, s, b):
      for g in range(nidx // _LANES):
        iv = idxr_v[pl.ds(s * slot + g * _LANES, _LANES)]
        wv = wr_v[pl.ds(s * slot + g * _LANES, _LANES)]
        hf = (iv & 1).astype(jnp.float32)
        pairs_v[pl.ds(b * wslot + g * _LANES, _LANES)] = (
            lax.shift_right_logical(iv, 1))
        wlo_v[pl.ds(b * wslot + g * _LANES, _LANES)] = wv * (1.0 - hf)
        whi_v[pl.ds(b * wslot + g * _LANES, _LANES)] = wv * hf

    def gather_copies(b):
      return [
          pltpu.make_async_copy(
              table_hbm.at[pairs_v.at[pl.ds(b * wslot + o, n)]],
              rows_v.at[b, pl.ds(o, n)], gsems[b])
          for o, n in splits
      ]

    def issue_gather(b):
      for d in gather_copies(b):
        d.start()

    def drain_gather(b):
      for d in gather_copies(b):
        d.wait()

    for c0 in range(niw):
      issue_iw(c0, c0)
    for b in range(nbuf):
      wait_iw(b, b)
      prep(b, b, b)
      issue_gather(b)

    def compute(c, b):
      def bag(k, carry):
        base = k * Lb
        acc = [jnp.zeros((_LANES,), jnp.float32) for _ in range(nq)]
        for g in range(ngrp):
          nrows = min(_LANES, Lb - g * _LANES)
          lo = wlo_v[pl.ds(b * wslot + base + g * _LANES, _LANES)]
          hi = whi_v[pl.ds(b * wslot + base + g * _LANES, _LANES)]
          for j2 in range(nrows):
            lv = jnp.full((_LANES,), lo[j2])
            hv = jnp.full((_LANES,), hi[j2])
            r = base + g * _LANES + j2
            for q in range(nq):
              acc[q] = (acc[q]
                        + lv * rows_v[b, r, pl.ds(q * _LANES, _LANES)]
                        + hv * rows_v[b, r, pl.ds(D + q * _LANES, _LANES)])
        for q in range(nq):
          out_v[k, pl.ds(q * _LANES, _LANES)] = acc[q]
        return carry

      lax.fori_loop(0, cb, bag, 0)
      pltpu.sync_copy(out_v, out_hbm.at[pl.ds(bag0 + c * cb, cb)])

    assert nchunk % niw == 0 and niw == 2 * nbuf

    def outer(i, carry):
      for j in range(niw):
        b = j % nbuf
        c = i * niw + j
        drain_gather(b)
        compute(c, b)
        c2 = c + nbuf

        @pl.when(c2 < nchunk)
        def _():
          wait_iw(c2, (j + nbuf) % niw)
          prep(c2, (j + nbuf) % niw, b)
          issue_gather(b)

        c4 = c + niw

        @pl.when(c4 < nchunk)
        def _():
          issue_iw(c4, j)
      return carry

    lax.fori_loop(0, nchunk // niw, outer, 0)

  fn = pl.kernel(
      body,
      out_type=jax.ShapeDtypeStruct((B, D), jnp.float32),
      mesh=mesh,
      scratch_types=[
          pltpu.VMEM((niw * slot,), jnp.int32),
          pltpu.VMEM((niw * slot,), jnp.float32),
          pltpu.VMEM((nbuf * wslot,), jnp.int32),
          pltpu.VMEM((nbuf * wslot,), jnp.float32),
          pltpu.VMEM((nbuf * wslot,), jnp.float32),
          pltpu.VMEM((nbuf, nidx, 2 * D), jnp.float32),
          pltpu.VMEM((cb, D), jnp.float32),
      ] + [pltpu.SemaphoreType.DMA] * (nbuf + niw),
      compiler_params=pltpu.CompilerParams(use_tc_tiling_on_sc=True),
      interpret=interpret,
  )
  return fn


def _make_sc_bag_sum_linear(B, Lb, D, table_rows, *, cb=2, nbuf=2, interpret=False):
  assert B % _NW == 0
  bpw = B // _NW
  assert bpw % cb == 0
  nchunk = bpw // cb
  assert nchunk % nbuf == 0
  assert D % _LANES == 0
  nq = D // _LANES

  mesh = plsc.VectorSubcoreMesh(
      core_axis_name="c", subcore_axis_name="s",
      num_cores=_NUM_CORES, num_subcores=_NUM_SUBCORES)

  def body(idx_hbm, w_hbm, table_hbm, out_hbm,
           idx_v, w_v, rows_v, out_v, *sems):
    cid = lax.axis_index("c")
    sid = lax.axis_index("s")
    wid = sid * _NUM_CORES + cid
    bag0 = wid * bpw

    pltpu.sync_copy(idx_hbm.at[pl.ds(bag0, bpw)], idx_v)
    pltpu.sync_copy(w_hbm.at[pl.ds(bag0 * Lb, bpw * Lb)],
                    w_v.at[pl.ds(0, bpw * Lb)])

    def start(chunk, b):
      for k in range(cb):
        bag = chunk * cb + k
        pltpu.async_copy(
            table_hbm.at[idx_v.at[bag]],
            rows_v.at[b, pl.ds(k * Lb, Lb)],
            sems[b])

    def drain(chunk, b):
      for k in range(cb):
        bag = chunk * cb + k
        pltpu.make_async_copy(
            table_hbm.at[idx_v.at[bag]],
            rows_v.at[b, pl.ds(k * Lb, Lb)],
            sems[b]).wait()

    for b in range(nbuf):
      start(b, b)

    ngrp = (Lb + _LANES - 1) // _LANES

    def outer(i, carry):
      for b in range(nbuf):
        chunk = i * nbuf + b
        drain(chunk, b)
        for k in range(cb):
          woff = (chunk * cb + k) * Lb
          acc = [jnp.zeros((_LANES,), jnp.float32) for _ in range(nq)]
          for g in range(ngrp):
            nrows = min(_LANES, Lb - g * _LANES)
            wvec = w_v[pl.ds(woff + g * _LANES, _LANES)]
            for j2 in range(nrows):
              wv = jnp.full((_LANES,), wvec[j2])
              r = k * Lb + g * _LANES + j2
              for q in range(nq):
                acc[q] = acc[q] + wv * rows_v[b, r, pl.ds(q * _LANES, _LANES)]
          for q in range(nq):
            out_v[k, pl.ds(q * _LANES, _LANES)] = acc[q]
        pltpu.sync_copy(out_v, out_hbm.at[pl.ds(bag0 + chunk * cb, cb)])
        nxt = chunk + nbuf

        @pl.when(nxt < nchunk)
        def _():
          start(nxt, b)
      return carry

    lax.fori_loop(0, nchunk // nbuf, outer, 0)

  fn = pl.kernel(
      body,
      out_type=jax.ShapeDtypeStruct((B, D), jnp.float32),
      mesh=mesh,
      scratch_types=[
          pltpu.VMEM((bpw, Lb), jnp.int32),
          pltpu.VMEM((bpw * Lb + _LANES,), jnp.float32),
          pltpu.VMEM((nbuf, cb * Lb, D), jnp.float32),
          pltpu.VMEM((cb, D), jnp.float32),
      ] + [pltpu.SemaphoreType.DMA] * nbuf,
      compiler_params=pltpu.CompilerParams(use_tc_tiling_on_sc=False),
      interpret=interpret,
  )
  return fn


def _make_tc_format(V, E, VA, *, blkc=2048, interpret=False):
  assert VA % blkc == 0 and blkc % 128 == 0
  nblk = VA // blkc
  last_blk = (V - 1) // blkc

  def body(a_ref, b_ref, o_ref):
    o_ref[:, pl.ds(0, E)] = jnp.transpose(a_ref[...])
    o_ref[:, pl.ds(E, E)] = jnp.transpose(b_ref[...])

  return pl.pallas_call(
      body,
      grid=(nblk,),
      in_specs=[
          pl.BlockSpec((E, blkc), lambda j: (0, j)),
          pl.BlockSpec((E, blkc),
                       lambda j: (0, jnp.minimum(j + nblk, last_blk))),
      ],
      out_specs=pl.BlockSpec((blkc, 2 * E), lambda j: (j, 0)),
      out_shape=jax.ShapeDtypeStruct((VA, 2 * E), jnp.float32),
      interpret=interpret,
  )


def _make_tc_mlp(B, D, E, *, blk=1024, interpret=False):
  assert B % blk == 0

  def body(x_ref, w1_ref, b1_ref, w2_ref, b2_ref, o_ref):
    x = x_ref[...]
    s = jnp.sum(x * x, axis=1, keepdims=True)
    x = x / jnp.maximum(jnp.sqrt(s), 1e-12)
    h = lax.dot_general(x, w1_ref[...], (((1,), (1,)), ((), ())),
                        preferred_element_type=jnp.float32) + b1_ref[...]
    h = jnp.where(h >= 0, h, 0.01 * h)
    o_ref[...] = lax.dot_general(h, w2_ref[...], (((1,), (1,)), ((), ())),
                                 preferred_element_type=jnp.float32) + b2_ref[...]

  grid = (B // blk,)
  return pl.pallas_call(
      body,
      grid=grid,
      in_specs=[
          pl.BlockSpec((blk, E), lambda i: (i, 0)),
          pl.BlockSpec((D, E), lambda i: (0, 0)),
          pl.BlockSpec((1, D), lambda i: (0, 0)),
          pl.BlockSpec((D, D), lambda i: (0, 0)),
          pl.BlockSpec((1, D), lambda i: (0, 0)),
      ],
      out_specs=pl.BlockSpec((blk, D), lambda i: (i, 0)),
      out_shape=jax.ShapeDtypeStruct((B, D), jnp.float32),
      interpret=interpret,
  )


@jax.jit
def kernel(indices, offsets, weights, base_emb, W1, b1, W2, b2):
  del offsets
  B = 16384
  Lb = 50
  V, E = base_emb.shape
  D = W1.shape[0]
  VA = 512000
  fmt = _make_tc_format(V, E, VA)
  table_lin = fmt(base_emb.T, base_emb.T).reshape(2 * VA, E)
  idxp = jnp.where(indices < VA, 2 * indices, 2 * (indices - VA) + 1)
  idx2d = idxp.reshape(B, Lb)
  sc = _make_sc_bag_sum_linear(B, Lb, E, 2 * VA)
  bag_sums = sc(idx2d, weights, table_lin)
  mlp = _make_tc_mlp(B, D, E)
  return mlp(bag_sums, W1, b1.reshape(1, D), W2, b2.reshape(1, D))


# speedup vs baseline: 4.3126x; 1.1706x over previous
import functools

import jax
import jax.numpy as jnp
from jax import lax
from jax.experimental import pallas as pl
from jax.experimental.pallas import tpu as pltpu
from jax.experimental.pallas import tpu_sc as plsc

_NUM_CORES = 2
_NUM_SUBCORES = 16
_NW = _NUM_CORES * _NUM_SUBCORES
_LANES = 16


def _make_sc_bag_sum(B, Lb, D, table_rows, va, *, cb=4, nbuf=2,
                     interpret=False):
  assert B % _NW == 0
  bpw = B // _NW
  assert bpw % cb == 0
  nchunk = bpw // cb
  assert nchunk % nbuf == 0
  assert D % _LANES == 0
  nq = D // _LANES
  ngrp = (Lb + _LANES - 1) // _LANES
  nnz_w = bpw * Lb
  offs = []
  o = 0
  while o + _LANES < Lb:
    offs.append(o)
    o += _LANES
  offs.append(Lb - _LANES)

  mesh = plsc.VectorSubcoreMesh(
      core_axis_name="c", subcore_axis_name="s",
      num_cores=_NUM_CORES, num_subcores=_NUM_SUBCORES)

  def body(idx_hbm, w_hbm, table_hbm, out_hbm,
           idx_v, w_v, gidx_v, rows_v, out_v, *sems):
    cid = lax.axis_index("c")
    sid = lax.axis_index("s")
    wid = sid * _NUM_CORES + cid
    gbase = wid * nnz_w
    bag0 = wid * bpw

    pltpu.sync_copy(idx_hbm.at[pl.ds(gbase, nnz_w)],
                    idx_v.at[pl.ds(0, nnz_w)])
    pltpu.sync_copy(w_hbm.at[pl.ds(gbase, nnz_w)],
                    w_v.at[pl.ds(0, nnz_w)])

    def prep(chunk, b):
      for k in range(cb):
        woff = (chunk * cb + k) * Lb
        for o in offs:
          iv = idx_v[pl.ds(woff + o, _LANES)]
          gidx_v[b, k, pl.ds(o, _LANES)] = jnp.where(
              iv < va, iv * 2, (iv - va) * 2 + 1)

    def gather_copies(b):
      return [
          pltpu.make_async_copy(
              table_hbm.at[gidx_v.at[b, k]],
              rows_v.at[b, pl.ds(k * Lb, Lb)],
              sems[b])
          for k in range(cb)
      ]

    def start(chunk, b):
      prep(chunk, b)
      for d in gather_copies(b):
        d.start()

    def drain(b):
      for d in gather_copies(b):
        d.wait()

    for b in range(nbuf):
      start(b, b)

    def outer(i, carry):
      for b in range(nbuf):
        chunk = i * nbuf + b
        drain(b)
        for k in range(cb):
          woff = (chunk * cb + k) * Lb
          acc = [jnp.zeros((_LANES,), jnp.float32) for _ in range(nq)]
          for g in range(ngrp):
            nrows = min(_LANES, Lb - g * _LANES)
            wvec = w_v[pl.ds(woff + g * _LANES, _LANES)]
            for j2 in range(nrows):
              wv = jnp.full((_LANES,), wvec[j2])
              r = k * Lb + g * _LANES + j2
              for q in range(nq):
                acc[q] = acc[q] + wv * rows_v[b, r, pl.ds(q * _LANES, _LANES)]
          for q in range(nq):
            out_v[k, pl.ds(q * _LANES, _LANES)] = acc[q]
        pltpu.sync_copy(out_v, out_hbm.at[pl.ds(bag0 + chunk * cb, cb)])
        nxt = chunk + nbuf

        @pl.when(nxt < nchunk)
        def _():
          start(nxt, b)
      return carry

    lax.fori_loop(0, nchunk // nbuf, outer, 0)

  fn = pl.kernel(
      body,
      out_type=jax.ShapeDtypeStruct((B, D), jnp.float32),
      mesh=mesh,
      scratch_types=[
          pltpu.VMEM((nnz_w + _LANES,), jnp.int32),
          pltpu.VMEM((nnz_w + _LANES,), jnp.float32),
          pltpu.VMEM((nbuf, cb, Lb), jnp.int32),
          pltpu.VMEM((nbuf, cb * Lb, D), jnp.float32),
          pltpu.VMEM((cb, D), jnp.float32),
      ] + [pltpu.SemaphoreType.DMA] * nbuf,
      compiler_params=pltpu.CompilerParams(use_tc_tiling_on_sc=False),
      interpret=interpret,
  )
  return fn


def _make_tc_format(V, E, VA, *, blkc=4096, interpret=False):
  assert VA % blkc == 0 and blkc % 128 == 0
  nblk = VA // blkc
  last_blk = (V - 1) // blkc

  def body(a_ref, b_ref, o_ref):
    o_ref[:, pl.ds(0, E)] = jnp.transpose(a_ref[...])
    o_ref[:, pl.ds(E, E)] = jnp.transpose(b_ref[...])

  return pl.pallas_call(
      body,
      grid=(nblk,),
      in_specs=[
          pl.BlockSpec((E, blkc), lambda j: (0, j)),
          pl.BlockSpec((E, blkc),
                       lambda j: (0, jnp.minimum(j + nblk, last_blk))),
      ],
      out_specs=pl.BlockSpec((blkc, 2 * E), lambda j: (j, 0)),
      out_shape=jax.ShapeDtypeStruct((VA, 2 * E), jnp.float32),
      interpret=interpret,
  )


def _make_tc_mlp(B, D, E, *, blk=1024, interpret=False):
  assert B % blk == 0

  def body(x_ref, w1_ref, b1_ref, w2_ref, b2_ref, o_ref):
    x = x_ref[...]
    s = jnp.sum(x * x, axis=1, keepdims=True)
    x = x / jnp.maximum(jnp.sqrt(s), 1e-12)
    ht = lax.dot_general(w1_ref[...], x, (((1,), (1,)), ((), ())),
                         preferred_element_type=jnp.float32) + b1_ref[...]
    ht = jnp.where(ht >= 0, ht, 0.01 * ht)
    o_ref[...] = lax.dot_general(w2_ref[...], ht, (((1,), (0,)), ((), ())),
                                 preferred_element_type=jnp.float32) + b2_ref[...]

  grid = (B // blk,)
  return pl.pallas_call(
      body,
      grid=grid,
      in_specs=[
          pl.BlockSpec((blk, E), lambda i: (i, 0)),
          pl.BlockSpec((D, E), lambda i: (0, 0)),
          pl.BlockSpec((D, 1), lambda i: (0, 0)),
          pl.BlockSpec((D, D), lambda i: (0, 0)),
          pl.BlockSpec((D, 1), lambda i: (0, 0)),
      ],
      out_specs=pl.BlockSpec((D, blk), lambda i: (0, i)),
      out_shape=jax.ShapeDtypeStruct((D, B), jnp.float32),
      interpret=interpret,
  )


@jax.jit
def kernel(indices, offsets, weights, base_emb, W1, b1, W2, b2):
  del offsets
  B = 16384
  Lb = 50
  V, E = base_emb.shape
  D = W1.shape[0]
  VA = 512000
  fmt = _make_tc_format(V, E, VA)
  table_lin = fmt(base_emb.T, base_emb.T).reshape(2 * VA, E)
  sc = _make_sc_bag_sum(B, Lb, E, 2 * VA, VA)
  bag_sums = sc(indices, weights, table_lin)
  mlp = _make_tc_mlp(B, D, E)
  yt = mlp(bag_sums, W1, b1.reshape(D, 1), W2, b2.reshape(D, 1))
  return yt.T
